# baseline (speedup 1.0000x reference)
import functools

import jax
import jax.numpy as jnp
from jax import lax
from jax.experimental import pallas as pl
from jax.experimental.pallas import tpu as pltpu
from jax.experimental.pallas import tpu_sc as plsc

N_ATOMS = 100000
NUM_EMB = 84
EMB = 128
UNIT = 128
NC, NS = 2, 16
NW = NC * NS
UNITS_PER_W = 25
ROWS_PER_W = UNITS_PER_W * UNIT
PAD_N = NW * ROWS_PER_W
N_MACRO = (UNITS_PER_W + 1) // 2
RING = 3
LAM = 2
LAST_ROW = N_ATOMS - UNIT


@functools.lru_cache(maxsize=None)
def _build():
    mesh = plsc.VectorSubcoreMesh(core_axis_name="c", subcore_axis_name="s")

    @functools.partial(
        pl.kernel,
        out_type=jax.ShapeDtypeStruct((N_ATOMS, EMB), jnp.float32),
        mesh=mesh,
        scratch_types=[
            pltpu.VMEM((ROWS_PER_W,), jnp.int32),
            pltpu.VMEM_SHARED((NUM_EMB, EMB), jnp.float32),
        ]
        + [pltpu.VMEM((2 * UNIT, EMB), jnp.float32)] * RING
        + [pltpu.SemaphoreType.DMA] * (2 * RING),
    )
    def emb(z_hbm, table_hbm, out_hbm, slab, table_sp, *rest):
        bufs = rest[:RING]
        gsems = rest[RING : 2 * RING]
        wsems = rest[2 * RING :]

        wid = lax.axis_index("s") * NC + lax.axis_index("c")
        row0 = pl.multiple_of(wid * ROWS_PER_W, ROWS_PER_W)

        @pl.when(lax.axis_index("s") == 0)
        def _():
            pltpu.sync_copy(table_hbm, table_sp)

        pltpu.sync_copy(z_hbm.at[pl.ds(row0, ROWS_PER_W)], slab)
        plsc.subcore_barrier()

        def units_of(m):
            return [2 * m] if m == N_MACRO - 1 else [2 * m, 2 * m + 1]

        def full_desc(m):
            return pltpu.make_async_copy(
                bufs[m % RING],
                out_hbm.at[pl.ds(pl.multiple_of(row0 + 2 * m * UNIT, 8), 2 * UNIT)],
                wsems[m % RING],
            )

        def single_desc(m):
            rb = jnp.minimum(row0 + 2 * m * UNIT, LAST_ROW)
            return pltpu.make_async_copy(
                bufs[m % RING].at[pl.ds(0, UNIT)],
                out_hbm.at[pl.ds(rb, UNIT)],
                wsems[m % RING],
            )

        def on_write(m, fn_full, fn_single):
            if m == N_MACRO - 1:
                @pl.when(row0 + 2 * m * UNIT < N_ATOMS)
                def _():
                    fn_single(m)
            else:
                full = row0 + (2 * m + 2) * UNIT <= N_ATOMS

                @pl.when(full)
                def _():
                    fn_full(m)

                @pl.when(jnp.logical_not(full) & (row0 + 2 * m * UNIT < N_ATOMS))
                def _():
                    fn_single(m)

        gds = {}

        def gather(m):
            for k, u in enumerate(units_of(m)):
                loff = pl.multiple_of(
                    jnp.minimum(row0 + u * UNIT, LAST_ROW) - row0, 8
                )
                gds[u] = pltpu.async_copy(
                    table_sp.at[slab.at[pl.ds(loff, UNIT)]],
                    bufs[m % RING].at[pl.ds(k * UNIT, UNIT)],
                    gsems[m % RING],
                )

        for t in range(N_MACRO + LAM):
            if t < N_MACRO:
                if t >= RING:
                    on_write(t - RING,
                             lambda m: full_desc(m).wait(),
                             lambda m: single_desc(m).wait())
                gather(t)
            c = t - LAM
            if c >= 0:
                for u in units_of(c):
                    gds[u].wait()
                on_write(c,
                         lambda m: full_desc(m).start(),
                         lambda m: single_desc(m).start())

        for p in range(max(0, N_MACRO - RING), N_MACRO):
            on_write(p,
                     lambda m: full_desc(m).wait(),
                     lambda m: single_desc(m).wait())

    return emb


def kernel(Z, table):
    z = jnp.pad(Z.astype(jnp.int32), (0, PAD_N - N_ATOMS))
    return _build()(z, table)


# speedup vs baseline: 1.0194x; 1.0194x over previous
import functools

import jax
import jax.numpy as jnp
from jax import lax
from jax.experimental import pallas as pl
from jax.experimental.pallas import tpu as pltpu
from jax.experimental.pallas import tpu_sc as plsc

N_ATOMS = 100000
NUM_EMB = 84
EMB = 128
UNIT = 128
N_UNITS = (N_ATOMS + UNIT - 1) // UNIT
NC, NS = 2, 16
NW = NC * NS
UNITS_PER_W = (N_UNITS + NW - 1) // NW
ROWS_PER_W = UNITS_PER_W * UNIT
PAD_N = NW * ROWS_PER_W
NBUF = 5
LOOKAHEAD = 3


@functools.lru_cache(maxsize=None)
def _build():
    mesh = plsc.VectorSubcoreMesh(core_axis_name="c", subcore_axis_name="s")

    @functools.partial(
        pl.kernel,
        out_type=jax.ShapeDtypeStruct((N_ATOMS, EMB), jnp.float32),
        mesh=mesh,
        scratch_types=[
            pltpu.VMEM((ROWS_PER_W,), jnp.int32),
            pltpu.VMEM_SHARED((NUM_EMB, EMB), jnp.float32),
        ]
        + [pltpu.VMEM((UNIT, EMB), jnp.float32)] * NBUF
        + [pltpu.SemaphoreType.DMA] * (2 * NBUF),
    )
    def emb(z_hbm, table_hbm, out_hbm, slab, table_sp, *rest):
        bufs = rest[:NBUF]
        gsems = rest[NBUF : 2 * NBUF]
        wsems = rest[2 * NBUF :]

        wid = lax.axis_index("s") * NC + lax.axis_index("c")
        row0 = pl.multiple_of(wid * ROWS_PER_W, ROWS_PER_W)

        @pl.when(lax.axis_index("s") == 0)
        def _():
            pltpu.sync_copy(table_hbm, table_sp)

        pltpu.sync_copy(z_hbm.at[pl.ds(row0, ROWS_PER_W)], slab)
        plsc.subcore_barrier()

        def unit_row(u):
            return jnp.minimum((row0 + u * UNIT), N_ATOMS - UNIT)

        def write_desc(u):
            rb = unit_row(u)
            return pltpu.make_async_copy(
                bufs[u % NBUF], out_hbm.at[pl.ds(rb, UNIT)], wsems[u % NBUF]
            )

        gds = {}
        for t in range(UNITS_PER_W + LOOKAHEAD):
            if t < UNITS_PER_W:
                prev = t - NBUF
                if prev >= 0:
                    @pl.when(row0 + prev * UNIT < N_ATOMS)
                    def _(prev=prev):
                        write_desc(prev).wait()

                loff = pl.multiple_of(unit_row(t) - row0, 8)
                gds[t] = pltpu.async_copy(
                    table_sp.at[slab.at[pl.ds(loff, UNIT)]],
                    bufs[t % NBUF],
                    gsems[t % NBUF],
                )
            v = t - LOOKAHEAD
            if v >= 0:
                gds[v].wait()

                @pl.when(row0 + v * UNIT < N_ATOMS)
                def _(v=v):
                    write_desc(v).start()

        for p in range(max(0, UNITS_PER_W - NBUF), UNITS_PER_W):
            @pl.when(row0 + p * UNIT < N_ATOMS)
            def _(p=p):
                write_desc(p).wait()

    return emb


def kernel(Z, table):
    z = jnp.pad(Z.astype(jnp.int32), (0, PAD_N - N_ATOMS))
    return _build()(z, table)


# speedup vs baseline: 1.0357x; 1.0160x over previous
import functools

import jax
import jax.numpy as jnp
from jax import lax
from jax.experimental import pallas as pl
from jax.experimental.pallas import tpu as pltpu
from jax.experimental.pallas import tpu_sc as plsc

N_ATOMS = 100000
NUM_EMB = 84
EMB = 128
UNIT = 128
N_UNITS = (N_ATOMS + UNIT - 1) // UNIT
NC, NS = 2, 16
NW = NC * NS
UNITS_PER_W = (N_UNITS + NW - 1) // NW
ROWS_PER_W = UNITS_PER_W * UNIT
PAD_N = NW * ROWS_PER_W
NBUF = 5
LOOKAHEAD = 3


@functools.lru_cache(maxsize=None)
def _build():
    mesh = plsc.VectorSubcoreMesh(core_axis_name="c", subcore_axis_name="s")

    @functools.partial(
        pl.kernel,
        out_type=jax.ShapeDtypeStruct((N_ATOMS, EMB), jnp.float32),
        mesh=mesh,
        scratch_types=[
            pltpu.VMEM((ROWS_PER_W,), jnp.int32),
            pltpu.VMEM_SHARED((NUM_EMB, EMB), jnp.float32),
        ]
        + [pltpu.VMEM((UNIT, EMB), jnp.float32)] * NBUF
        + [pltpu.SemaphoreType.DMA] * (2 * NBUF + 1),
    )
    def emb(z_hbm, table_hbm, out_hbm, slab, table_sp, *rest):
        bufs = rest[:NBUF]
        gsems = rest[NBUF : 2 * NBUF]
        wsems = rest[2 * NBUF : 3 * NBUF]
        tsem = rest[3 * NBUF]

        wid = lax.axis_index("s") * NC + lax.axis_index("c")
        row0 = pl.multiple_of(wid * ROWS_PER_W, ROWS_PER_W)

        is_stager = lax.axis_index("s") == 0

        @pl.when(is_stager)
        def _():
            pltpu.make_async_copy(table_hbm, table_sp, tsem).start()

        pltpu.sync_copy(z_hbm.at[pl.ds(row0, ROWS_PER_W)], slab)

        @pl.when(is_stager)
        def _():
            pltpu.make_async_copy(table_hbm, table_sp, tsem).wait()

        plsc.subcore_barrier()

        def unit_row(u):
            return jnp.minimum((row0 + u * UNIT), N_ATOMS - UNIT)

        def write_desc(u):
            rb = unit_row(u)
            return pltpu.make_async_copy(
                bufs[u % NBUF], out_hbm.at[pl.ds(rb, UNIT)], wsems[u % NBUF]
            )

        gds = {}
        for t in range(UNITS_PER_W + LOOKAHEAD):
            if t < UNITS_PER_W:
                prev = t - NBUF
                if prev >= 0:
                    @pl.when(row0 + prev * UNIT < N_ATOMS)
                    def _(prev=prev):
                        write_desc(prev).wait()

                loff = pl.multiple_of(unit_row(t) - row0, 8)
                gds[t] = pltpu.async_copy(
                    table_sp.at[slab.at[pl.ds(loff, UNIT)]],
                    bufs[t % NBUF],
                    gsems[t % NBUF],
                )
            v = t - LOOKAHEAD
            if v >= 0:
                gds[v].wait()

                @pl.when(row0 + v * UNIT < N_ATOMS)
                def _(v=v):
                    write_desc(v).start()

        for p in range(max(0, UNITS_PER_W - NBUF), UNITS_PER_W):
            @pl.when(row0 + p * UNIT < N_ATOMS)
            def _(p=p):
                write_desc(p).wait()

    return emb


def kernel(Z, table):
    z = jnp.pad(Z.astype(jnp.int32), (0, PAD_N - N_ATOMS))
    return _build()(z, table)
